# baseline (speedup 1.0000x reference)
import functools

import jax
import jax.numpy as jnp
from jax import lax
from jax.experimental import pallas as pl
from jax.experimental.pallas import tpu as pltpu
from jax.experimental.pallas import tpu_sc as plsc

H = 7168
E = 256
TOPK = 8
N_GROUP = 8
TOPK_GROUP = 4
GROUP_SIZE = E // N_GROUP
SCALE = 2.5

BT = 512
NW = 32
TL = 128
TILES_PER_W = 16384 // (NW * TL)
L = 16

NEG_INF = float("-inf")


def _mask_body(h_ref, w_ref, b_ref, m_ref):
    logits = jax.lax.dot_general(
        w_ref[...], h_ref[...],
        (((1,), (1,)), ((), ())),
        preferred_element_type=jnp.float32)
    scores = jax.nn.sigmoid(logits)
    sc = scores + b_ref[...]
    bt = sc.shape[1]

    gs_rows = []
    for g in range(N_GROUP):
        x = sc[g * GROUP_SIZE:(g + 1) * GROUP_SIZE, :]
        it = jax.lax.broadcasted_iota(jnp.int32, x.shape, 0)
        m1 = jnp.max(x, axis=0, keepdims=True)
        first = jnp.min(jnp.where(x == m1, it, GROUP_SIZE * 2),
                        axis=0, keepdims=True)
        m2 = jnp.max(jnp.where(it == first, NEG_INF, x),
                     axis=0, keepdims=True)
        gs_rows.append(m1 + m2)
    gs = jnp.concatenate(gs_rows, axis=0)

    git = jax.lax.broadcasted_iota(jnp.int32, gs.shape, 0)
    gmask = jnp.zeros(gs.shape, dtype=jnp.float32)
    work = gs
    for _ in range(TOPK_GROUP):
        m = jnp.max(work, axis=0, keepdims=True)
        sel = jnp.min(jnp.where(work == m, git, N_GROUP * 2),
                      axis=0, keepdims=True)
        hit = git == sel
        gmask = jnp.where(hit, 1.0, gmask)
        work = jnp.where(hit, NEG_INF, work)

    em = jnp.concatenate(
        [jnp.broadcast_to(gmask[g:g + 1, :], (GROUP_SIZE, bt))
         for g in range(N_GROUP)], axis=0)
    masked = jnp.where(em > 0, sc, NEG_INF)

    for t in range(BT // TL):
        m_ref[0, t] = masked[:, t * TL:(t + 1) * TL]


def _masked_scores(hidden_states, weight, bias_col):
    t = hidden_states.shape[0]
    grid = (t // BT,)
    return pl.pallas_call(
        _mask_body,
        grid=grid,
        in_specs=[
            pl.BlockSpec((BT, H), lambda i: (i, 0)),
            pl.BlockSpec((E, H), lambda i: (0, 0)),
            pl.BlockSpec((E, 1), lambda i: (0, 0)),
        ],
        out_specs=pl.BlockSpec((1, BT // TL, E, TL),
                               lambda i: (i, 0, 0, 0)),
        out_shape=jax.ShapeDtypeStruct((t // BT, BT // TL, E, TL),
                                       jnp.float32),
    )(hidden_states, weight, bias_col)


_GDN = lax.GatherDimensionNumbers(
    offset_dims=(), collapsed_slice_dims=(0,), start_index_map=(0,))


def _vreg_gather(src, lo):
    return lax.gather(src, lo.reshape(L, 1), _GDN, (1,),
                      mode=lax.GatherScatterMode.PROMISE_IN_BOUNDS)


def _sc_topk_body(tiles_per_w, m_hbm, b_hbm, idx_hbm, w_hbm,
                  tile_v, bias_v, idx_stage, w_stage):
    wid = lax.axis_index("s") * 2 + lax.axis_index("c")
    pltpu.sync_copy(b_hbm, bias_v)
    bias_regs = [bias_v[g, :] for g in range(E // L)]
    base = wid * tiles_per_w

    for t in range(tiles_per_w):
        pltpu.sync_copy(m_hbm.at[base + t], tile_v)
        for j in range(TL // L):
            sl = pl.ds(j * L, L)

            def insert(e, carry):
                vals = carry[:TOPK]
                idxs = carry[TOPK:]
                v = tile_v[e, sl]
                ei = jnp.full((L,), e, dtype=jnp.int32)
                new_vals, new_idxs = [], []
                c_prev = jnp.zeros((L,), dtype=jnp.bool_)
                for k in range(TOPK):
                    c = v > vals[k]
                    nv = jnp.where(c, jnp.where(c_prev, vals[k - 1], v),
                                   vals[k])
                    ni = jnp.where(c, jnp.where(c_prev, idxs[k - 1], ei),
                                   idxs[k])
                    new_vals.append(nv)
                    new_idxs.append(ni)
                    c_prev = c
                return tuple(new_vals) + tuple(new_idxs)

            init = (tuple(jnp.full((L,), NEG_INF, dtype=jnp.float32)
                          for _ in range(TOPK))
                    + tuple(jnp.zeros((L,), dtype=jnp.int32)
                            for _ in range(TOPK)))
            carry = lax.fori_loop(0, E, insert, init, unroll=4)
            vals = carry[:TOPK]
            idxs = carry[TOPK:]

            ws = []
            for k in range(TOPK):
                lo = jnp.bitwise_and(idxs[k], L - 1)
                hi = lax.shift_right_logical(idxs[k], 4)
                bk = jnp.zeros((L,), jnp.float32)
                for g in range(E // L):
                    cand = _vreg_gather(bias_regs[g], lo)
                    bk = jnp.where(hi == g, cand, bk)
                ws.append(vals[k] - bk)
            denom = ws[0]
            for k in range(1, TOPK):
                denom = denom + ws[k]
            scale = SCALE / (denom + 1e-20)
            for k in range(TOPK):
                idx_stage[k, sl] = idxs[k]
                w_stage[k, sl] = ws[k] * scale

        pltpu.sync_copy(idx_stage, idx_hbm.at[base + t])
        pltpu.sync_copy(w_stage, w_hbm.at[base + t])


def _sc_topk(masked4, bias):
    nt = masked4.shape[0]
    mesh = plsc.VectorSubcoreMesh(core_axis_name="c", subcore_axis_name="s")
    f = pl.kernel(
        functools.partial(_sc_topk_body, nt // NW),
        mesh=mesh,
        out_type=[
            jax.ShapeDtypeStruct((nt, TOPK, TL), jnp.int32),
            jax.ShapeDtypeStruct((nt, TOPK, TL), jnp.float32),
        ],
        scratch_types=[
            pltpu.VMEM((E, TL), jnp.float32),
            pltpu.VMEM((E // L, L), jnp.float32),
            pltpu.VMEM((TOPK, TL), jnp.int32),
            pltpu.VMEM((TOPK, TL), jnp.float32),
        ],
    )
    return f(masked4, bias)


NCHUNK = 4


@jax.jit
def kernel(hidden_states, weight, e_score_correction_bias):
    t = hidden_states.shape[0]
    tc = t // NCHUNK
    bias_col = e_score_correction_bias.reshape(E, 1)
    bias_sq = e_score_correction_bias.reshape(E // L, L)
    idx_parts, w_parts = [], []
    for c in range(NCHUNK):
        h = lax.slice_in_dim(hidden_states, c * tc, (c + 1) * tc, axis=0)
        masked4 = _masked_scores(h, weight, bias_col)
        masked4 = masked4.reshape(tc // TL, E, TL)
        idx4, w4 = _sc_topk(masked4, bias_sq)
        idx_parts.append(idx4.transpose(0, 2, 1).reshape(tc, TOPK))
        w_parts.append(w4.transpose(0, 2, 1).reshape(tc, TOPK))
    idx = jnp.concatenate(idx_parts, axis=0)
    w = jnp.concatenate(w_parts, axis=0)
    return idx, w


# speedup vs baseline: 2.4588x; 2.4588x over previous
import functools

import jax
import jax.numpy as jnp
from jax import lax
from jax.experimental import pallas as pl
from jax.experimental.pallas import tpu as pltpu
from jax.experimental.pallas import tpu_sc as plsc

H = 7168
E = 256
TOPK = 8
N_GROUP = 8
TOPK_GROUP = 4
GROUP_SIZE = E // N_GROUP
SCALE = 2.5

BT = 512
NW = 32
TL = 128
L = 16
NC = N_GROUP * TOPK

NEG_INF = float("-inf")


def _mask_body(h_ref, w_ref, b_ref, v_ref, i_ref):
    logits = jax.lax.dot_general(
        w_ref[...], h_ref[...],
        (((1,), (1,)), ((), ())),
        preferred_element_type=jnp.float32)
    scores = jax.nn.sigmoid(logits)
    sc = scores + b_ref[...]
    bt = sc.shape[1]

    gs_rows = []
    for g in range(N_GROUP):
        x = sc[g * GROUP_SIZE:(g + 1) * GROUP_SIZE, :]
        it = jax.lax.broadcasted_iota(jnp.int32, x.shape, 0)
        m1 = jnp.max(x, axis=0, keepdims=True)
        first = jnp.min(jnp.where(x == m1, it, GROUP_SIZE * 2),
                        axis=0, keepdims=True)
        m2 = jnp.max(jnp.where(it == first, NEG_INF, x),
                     axis=0, keepdims=True)
        gs_rows.append(m1 + m2)
    gs = jnp.concatenate(gs_rows, axis=0)

    git = jax.lax.broadcasted_iota(jnp.int32, gs.shape, 0)
    gmask = jnp.zeros(gs.shape, dtype=jnp.float32)
    work = gs
    for _ in range(TOPK_GROUP):
        m = jnp.max(work, axis=0, keepdims=True)
        sel = jnp.min(jnp.where(work == m, git, N_GROUP * 2),
                      axis=0, keepdims=True)
        hit = git == sel
        gmask = jnp.where(hit, 1.0, gmask)
        work = jnp.where(hit, NEG_INF, work)

    val_rows, idx_rows = [], []
    for g in range(N_GROUP):
        x = sc[g * GROUP_SIZE:(g + 1) * GROUP_SIZE, :]
        it = jax.lax.broadcasted_iota(jnp.int32, x.shape, 0)
        gsel = gmask[g:g + 1, :] > 0
        for _ in range(TOPK):
            m = jnp.max(x, axis=0, keepdims=True)
            sel = jnp.min(jnp.where(x == m, it, GROUP_SIZE * 2),
                          axis=0, keepdims=True)
            x = jnp.where(it == sel, NEG_INF, x)
            val_rows.append(jnp.where(gsel, m, NEG_INF))
            idx_rows.append(sel + (g * GROUP_SIZE))
    vals = jnp.concatenate(val_rows, axis=0)
    idxs = jnp.concatenate(idx_rows, axis=0)

    for t in range(BT // TL):
        v_ref[0, t] = vals[:, t * TL:(t + 1) * TL]
        i_ref[0, t] = idxs[:, t * TL:(t + 1) * TL]


def _masked_scores(hidden_states, weight, bias_col):
    t = hidden_states.shape[0]
    grid = (t // BT,)
    return pl.pallas_call(
        _mask_body,
        grid=grid,
        in_specs=[
            pl.BlockSpec((BT, H), lambda i: (i, 0)),
            pl.BlockSpec((E, H), lambda i: (0, 0)),
            pl.BlockSpec((E, 1), lambda i: (0, 0)),
        ],
        out_specs=[
            pl.BlockSpec((1, BT // TL, NC, TL), lambda i: (i, 0, 0, 0)),
            pl.BlockSpec((1, BT // TL, NC, TL), lambda i: (i, 0, 0, 0)),
        ],
        out_shape=[
            jax.ShapeDtypeStruct((t // BT, BT // TL, NC, TL), jnp.float32),
            jax.ShapeDtypeStruct((t // BT, BT // TL, NC, TL), jnp.int32),
        ],
    )(hidden_states, weight, bias_col)


_GDN = lax.GatherDimensionNumbers(
    offset_dims=(), collapsed_slice_dims=(0,), start_index_map=(0,))


def _vreg_gather(src, lo):
    return lax.gather(src, lo.reshape(L, 1), _GDN, (1,),
                      mode=lax.GatherScatterMode.PROMISE_IN_BOUNDS)


def _sc_topk_body(tiles_per_w, v_hbm, i_hbm, b_hbm, idx_hbm, w_hbm,
                  tile_v, tile_i, bias_v, idx_stage, w_stage):
    wid = lax.axis_index("s") * 2 + lax.axis_index("c")
    pltpu.sync_copy(b_hbm, bias_v)
    bias_regs = [bias_v[g, :] for g in range(E // L)]
    base = wid * tiles_per_w

    def tile_body(t, _):
        pltpu.sync_copy(v_hbm.at[base + t], tile_v)
        pltpu.sync_copy(i_hbm.at[base + t], tile_i)

        def slice_body(j, _):
            sl = pl.ds(j * L, L)

            def insert(e, carry):
                vals = carry[:TOPK]
                idxs = carry[TOPK:]
                v = tile_v[e, sl]
                ei = tile_i[e, sl]
                new_vals, new_idxs = [], []
                c_prev = jnp.zeros((L,), dtype=jnp.bool_)
                for k in range(TOPK):
                    c = v > vals[k]
                    nv = jnp.where(c, jnp.where(c_prev, vals[k - 1], v),
                                   vals[k])
                    ni = jnp.where(c, jnp.where(c_prev, idxs[k - 1], ei),
                                   idxs[k])
                    new_vals.append(nv)
                    new_idxs.append(ni)
                    c_prev = c
                return tuple(new_vals) + tuple(new_idxs)

            init = (tuple(jnp.full((L,), NEG_INF, dtype=jnp.float32)
                          for _ in range(TOPK))
                    + tuple(jnp.zeros((L,), dtype=jnp.int32)
                            for _ in range(TOPK)))
            carry = lax.fori_loop(0, NC, insert, init, unroll=4)
            vals = carry[:TOPK]
            idxs = carry[TOPK:]

            ws = []
            for k in range(TOPK):
                lo = jnp.bitwise_and(idxs[k], L - 1)
                hi = lax.shift_right_logical(idxs[k], 4)
                bk = jnp.zeros((L,), jnp.float32)
                for g in range(E // L):
                    cand = _vreg_gather(bias_regs[g], lo)
                    bk = jnp.where(hi == g, cand, bk)
                ws.append(vals[k] - bk)
            denom = ws[0]
            for k in range(1, TOPK):
                denom = denom + ws[k]
            scale = SCALE / (denom + 1e-20)
            for k in range(TOPK):
                idx_stage[k, sl] = idxs[k]
                w_stage[k, sl] = ws[k] * scale
            return 0

        lax.fori_loop(0, TL // L, slice_body, 0)
        pltpu.sync_copy(idx_stage, idx_hbm.at[base + t])
        pltpu.sync_copy(w_stage, w_hbm.at[base + t])
        return 0

    lax.fori_loop(0, tiles_per_w, tile_body, 0)


def _sc_topk(vals4, idxs4, bias):
    nt = vals4.shape[0]
    mesh = plsc.VectorSubcoreMesh(core_axis_name="c", subcore_axis_name="s")
    f = pl.kernel(
        functools.partial(_sc_topk_body, nt // NW),
        mesh=mesh,
        out_type=[
            jax.ShapeDtypeStruct((nt, TOPK, TL), jnp.int32),
            jax.ShapeDtypeStruct((nt, TOPK, TL), jnp.float32),
        ],
        scratch_types=[
            pltpu.VMEM((NC, TL), jnp.float32),
            pltpu.VMEM((NC, TL), jnp.int32),
            pltpu.VMEM((E // L, L), jnp.float32),
            pltpu.VMEM((TOPK, TL), jnp.int32),
            pltpu.VMEM((TOPK, TL), jnp.float32),
        ],
    )
    return f(vals4, idxs4, bias)


NCHUNK = 1


@jax.jit
def kernel(hidden_states, weight, e_score_correction_bias):
    t = hidden_states.shape[0]
    tc = t // NCHUNK
    bias_col = e_score_correction_bias.reshape(E, 1)
    bias_sq = e_score_correction_bias.reshape(E // L, L)
    idx_parts, w_parts = [], []
    for c in range(NCHUNK):
        h = lax.slice_in_dim(hidden_states, c * tc, (c + 1) * tc, axis=0)
        vals4, idxs4 = _masked_scores(h, weight, bias_col)
        vals4 = vals4.reshape(tc // TL, NC, TL)
        idxs4 = idxs4.reshape(tc // TL, NC, TL)
        idx4, w4 = _sc_topk(vals4, idxs4, bias_sq)
        idx_parts.append(idx4.transpose(0, 2, 1).reshape(tc, TOPK))
        w_parts.append(w4.transpose(0, 2, 1).reshape(tc, TOPK))
    idx = jnp.concatenate(idx_parts, axis=0)
    w = jnp.concatenate(w_parts, axis=0)
    return idx, w
